# baseline (speedup 1.0000x reference)
import functools

import jax
import jax.numpy as jnp
from jax import lax
from jax.experimental import pallas as pl
from jax.experimental.pallas import tpu as pltpu
from jax.experimental.pallas import tpu_sc as plsc

B = 16384
D = 64
NC = 2
NS = 16
NW = NC * NS
BPW = B // NW
GCH = 128
NG = BPW // GCH


def _gmf_body(uid_hbm, iid_hbm, ut_hbm, it_hbm, w_hbm, b_hbm, out_hbm,
              idx_u, idx_i, rows_u, rows_i, w_v, b_v, out_v, sem):
    wid = lax.axis_index("s") * NC + lax.axis_index("c")
    base = wid * BPW

    pltpu.sync_copy(uid_hbm.at[wid], idx_u)
    pltpu.sync_copy(iid_hbm.at[wid], idx_i)
    pltpu.sync_copy(w_hbm, w_v)
    pltpu.sync_copy(b_hbm, b_v)

    copies = []
    for j in range(NG):
        copies.append(pltpu.async_copy(
            ut_hbm.at[idx_u.at[j]], rows_u.at[pl.ds(j * GCH, GCH)], sem))
        copies.append(pltpu.async_copy(
            it_hbm.at[idx_i.at[j]], rows_i.at[pl.ds(j * GCH, GCH)], sem))
    for c in copies:
        c.wait()

    w0 = w_v[pl.ds(0, 16)]
    w1 = w_v[pl.ds(16, 16)]
    w2 = w_v[pl.ds(32, 16)]
    w3 = w_v[pl.ds(48, 16)]
    bvec = b_v[...]
    lane = lax.iota(jnp.int32, 16)

    def group(g, carry):
        vec = jnp.zeros((16,), jnp.float32)
        for k in range(16):
            r = g * 16 + k
            acc = rows_u[r, pl.ds(0, 16)] * rows_i[r, pl.ds(0, 16)] * w0
            acc += rows_u[r, pl.ds(16, 16)] * rows_i[r, pl.ds(16, 16)] * w1
            acc += rows_u[r, pl.ds(32, 16)] * rows_i[r, pl.ds(32, 16)] * w2
            acc += rows_u[r, pl.ds(48, 16)] * rows_i[r, pl.ds(48, 16)] * w3
            vec = jnp.where(lane == k, jnp.sum(acc), vec)
        out_v[pl.ds(g * 16, 16)] = vec + bvec
        return carry

    lax.fori_loop(0, BPW // 16, group, 0)

    pltpu.sync_copy(out_v, out_hbm.at[pl.ds(base, BPW)])


@jax.jit
def kernel(userID, itemID, user_table, item_table, W, b):
    uid3 = userID.reshape(NW, NG, GCH)
    iid3 = itemID.reshape(NW, NG, GCH)
    w1d = W.reshape(D)
    b16 = jnp.broadcast_to(b.astype(jnp.float32), (16,))

    mesh = plsc.VectorSubcoreMesh(core_axis_name="c", subcore_axis_name="s")
    f = pl.kernel(
        _gmf_body,
        mesh=mesh,
        compiler_params=pltpu.CompilerParams(
            needs_layout_passes=False, use_tc_tiling_on_sc=False),
        out_type=jax.ShapeDtypeStruct((B,), jnp.float32),
        scratch_types=[
            pltpu.VMEM((NG, GCH), jnp.int32),
            pltpu.VMEM((NG, GCH), jnp.int32),
            pltpu.VMEM((BPW, D), jnp.float32),
            pltpu.VMEM((BPW, D), jnp.float32),
            pltpu.VMEM((D,), jnp.float32),
            pltpu.VMEM((16,), jnp.float32),
            pltpu.VMEM((BPW,), jnp.float32),
            pltpu.SemaphoreType.DMA,
        ],
    )
    return f(uid3, iid3, user_table, item_table, w1d, b16)


# speedup vs baseline: 2.1687x; 2.1687x over previous
import functools

import jax
import jax.numpy as jnp
from jax import lax
from jax.experimental import pallas as pl
from jax.experimental.pallas import tpu as pltpu
from jax.experimental.pallas import tpu_sc as plsc

B = 16384
D = 64
NC = 2
NS = 16
NW = NC * NS
BPW = B // NW
C = 32
NCH = BPW // C


def _gmf_body(uid_hbm, iid_hbm, ut_hbm, it_hbm, w_hbm, b_hbm, out_hbm,
              idx_u, idx_i, tiles_u, tiles_i, w_v, b_v, out_v, sem_u, sem_i):
    wid = lax.axis_index("s") * NC + lax.axis_index("c")
    base = wid * BPW

    pltpu.sync_copy(uid_hbm.at[pl.ds(base, BPW)], idx_u)
    pltpu.sync_copy(iid_hbm.at[pl.ds(base, BPW)], idx_i)
    pltpu.sync_copy(w_hbm, w_v)
    pltpu.sync_copy(b_hbm, b_v)

    w0 = w_v[pl.ds(0, 16)]
    w1 = w_v[pl.ds(16, 16)]
    w2 = w_v[pl.ds(32, 16)]
    w3 = w_v[pl.ds(48, 16)]
    bvec = b_v[...]
    lane = lax.iota(jnp.int32, 16)

    def chunk(ch, carry):
        off = ch * C
        copies = []
        rus, ris = [], []
        for o in range(0, C, 16):
            uvec = idx_u[pl.ds(off + o, 16)]
            ivec = idx_i[pl.ds(off + o, 16)]
            tu_vec = lax.shift_right_logical(uvec, 3)
            ti_vec = lax.shift_right_logical(ivec, 3)
            ru_vec = uvec & 7
            ri_vec = ivec & 7
            for k in range(16):
                copies.append(pltpu.async_copy(
                    ut_hbm.at[tu_vec[k]], tiles_u.at[o + k], sem_u))
                copies.append(pltpu.async_copy(
                    it_hbm.at[ti_vec[k]], tiles_i.at[o + k], sem_i))
                rus.append(ru_vec[k])
                ris.append(ri_vec[k])
        for c in copies:
            c.wait()

        vec = jnp.zeros((16,), jnp.float32)
        for j in range(C):
            ru = rus[j]
            ri = ris[j]
            acc = tiles_u[j, ru, pl.ds(0, 16)] * tiles_i[j, ri, pl.ds(0, 16)] * w0
            acc += tiles_u[j, ru, pl.ds(16, 16)] * tiles_i[j, ri, pl.ds(16, 16)] * w1
            acc += tiles_u[j, ru, pl.ds(32, 16)] * tiles_i[j, ri, pl.ds(32, 16)] * w2
            acc += tiles_u[j, ru, pl.ds(48, 16)] * tiles_i[j, ri, pl.ds(48, 16)] * w3
            vec = jnp.where(lane == (j % 16), jnp.sum(acc), vec)
            if j % 16 == 15:
                out_v[pl.ds(off + j - 15, 16)] = vec + bvec
                vec = jnp.zeros((16,), jnp.float32)
        return carry

    lax.fori_loop(0, NCH, chunk, 0)

    pltpu.sync_copy(out_v, out_hbm.at[pl.ds(base, BPW)])


@jax.jit
def kernel(userID, itemID, user_table, item_table, W, b):
    ut3 = user_table.reshape(-1, 8, D)
    it3 = item_table.reshape(-1, 8, D)
    w1d = W.reshape(D)
    b16 = jnp.broadcast_to(b.astype(jnp.float32), (16,))

    mesh = plsc.VectorSubcoreMesh(core_axis_name="c", subcore_axis_name="s")
    f = pl.kernel(
        _gmf_body,
        mesh=mesh,
        compiler_params=pltpu.CompilerParams(needs_layout_passes=False),
        out_type=jax.ShapeDtypeStruct((B,), jnp.float32),
        scratch_types=[
            pltpu.VMEM((BPW,), jnp.int32),
            pltpu.VMEM((BPW,), jnp.int32),
            pltpu.VMEM((C, 8, D), jnp.float32),
            pltpu.VMEM((C, 8, D), jnp.float32),
            pltpu.VMEM((D,), jnp.float32),
            pltpu.VMEM((16,), jnp.float32),
            pltpu.VMEM((BPW,), jnp.float32),
            pltpu.SemaphoreType.DMA,
            pltpu.SemaphoreType.DMA,
        ],
    )
    return f(userID, itemID, ut3, it3, w1d, b16)


# speedup vs baseline: 2.2094x; 1.0187x over previous
import functools

import jax
import jax.numpy as jnp
from jax import lax
from jax.experimental import pallas as pl
from jax.experimental.pallas import tpu as pltpu
from jax.experimental.pallas import tpu_sc as plsc

B = 16384
D = 64
NC = 2
NS = 16
NW = NC * NS
BPW = B // NW
C = 16
NCH = BPW // C


def _gmf_body(uid_hbm, iid_hbm, ut_hbm, it_hbm, w_hbm, b_hbm, out_hbm,
              idx_u, idx_i, tu_a, tu_b, ti_a, ti_b, w_v, b_v, out_v,
              sem_u, sem_i):
    wid = lax.axis_index("s") * NC + lax.axis_index("c")
    base = wid * BPW

    pltpu.sync_copy(uid_hbm.at[pl.ds(base, BPW)], idx_u)
    pltpu.sync_copy(iid_hbm.at[pl.ds(base, BPW)], idx_i)
    pltpu.sync_copy(w_hbm, w_v)
    pltpu.sync_copy(b_hbm, b_v)

    w0 = w_v[pl.ds(0, 16)]
    w1 = w_v[pl.ds(16, 16)]
    w2 = w_v[pl.ds(32, 16)]
    w3 = w_v[pl.ds(48, 16)]
    bvec = b_v[...]
    lane = lax.iota(jnp.int32, 16)

    def issue(ch, bu, bi):
        uvec = lax.shift_right_logical(idx_u[pl.ds(ch * C, 16)], 3)
        ivec = lax.shift_right_logical(idx_i[pl.ds(ch * C, 16)], 3)
        for k in range(16):
            pltpu.async_copy(ut_hbm.at[uvec[k]], bu.at[k], sem_u)
            pltpu.async_copy(it_hbm.at[ivec[k]], bi.at[k], sem_i)

    def drain():
        pltpu.make_async_copy(ut_hbm.at[pl.ds(0, C)], tu_a, sem_u).wait()
        pltpu.make_async_copy(it_hbm.at[pl.ds(0, C)], ti_a, sem_i).wait()

    def compute(ch, bu, bi):
        ru_vec = idx_u[pl.ds(ch * C, 16)] & 7
        ri_vec = idx_i[pl.ds(ch * C, 16)] & 7
        vec = jnp.zeros((16,), jnp.float32)
        for k in range(16):
            ru = ru_vec[k]
            ri = ri_vec[k]
            acc = bu[k, ru, pl.ds(0, 16)] * bi[k, ri, pl.ds(0, 16)] * w0
            acc += bu[k, ru, pl.ds(16, 16)] * bi[k, ri, pl.ds(16, 16)] * w1
            acc += bu[k, ru, pl.ds(32, 16)] * bi[k, ri, pl.ds(32, 16)] * w2
            acc += bu[k, ru, pl.ds(48, 16)] * bi[k, ri, pl.ds(48, 16)] * w3
            vec = jnp.where(lane == k, jnp.sum(acc), vec)
        out_v[pl.ds(ch * C, 16)] = vec + bvec

    issue(0, tu_a, ti_a)

    def super_chunk(s, carry):
        ch = 2 * s
        issue(ch + 1, tu_b, ti_b)
        drain()
        compute(ch, tu_a, ti_a)
        issue((ch + 2) % NCH, tu_a, ti_a)
        drain()
        compute(ch + 1, tu_b, ti_b)
        return carry

    lax.fori_loop(0, NCH // 2, super_chunk, 0)
    drain()

    pltpu.sync_copy(out_v, out_hbm.at[pl.ds(base, BPW)])


@jax.jit
def kernel(userID, itemID, user_table, item_table, W, b):
    ut3 = user_table.reshape(-1, 8, D)
    it3 = item_table.reshape(-1, 8, D)
    w1d = W.reshape(D)
    b16 = jnp.broadcast_to(b.astype(jnp.float32), (16,))

    mesh = plsc.VectorSubcoreMesh(core_axis_name="c", subcore_axis_name="s")
    f = pl.kernel(
        _gmf_body,
        mesh=mesh,
        compiler_params=pltpu.CompilerParams(needs_layout_passes=False),
        out_type=jax.ShapeDtypeStruct((B,), jnp.float32),
        scratch_types=[
            pltpu.VMEM((BPW,), jnp.int32),
            pltpu.VMEM((BPW,), jnp.int32),
            pltpu.VMEM((C, 8, D), jnp.float32),
            pltpu.VMEM((C, 8, D), jnp.float32),
            pltpu.VMEM((C, 8, D), jnp.float32),
            pltpu.VMEM((C, 8, D), jnp.float32),
            pltpu.VMEM((D,), jnp.float32),
            pltpu.VMEM((16,), jnp.float32),
            pltpu.VMEM((BPW,), jnp.float32),
            pltpu.SemaphoreType.DMA,
            pltpu.SemaphoreType.DMA,
        ],
    )
    return f(userID, itemID, ut3, it3, w1d, b16)
